# baseline (speedup 1.0000x reference)
import functools
import jax
import jax.numpy as jnp
from jax import lax
from jax.experimental import pallas as pl
from jax.experimental.pallas import tpu as pltpu
from jax.experimental.pallas import tpu_sc as plsc

N = 100000
E = 1600000
F_IN = 128
H = 32

NC, NS, LANES = 2, 16, 16

NP_ROWS = 800
HALF = N // 2
ACC_ROWS = 50176
ZROWS = 784
EPT = E // NS
EPW = E // (NC * NS)
CK = 2000
CKA = 400
CKD = 2000

_MESH = plsc.VectorSubcoreMesh(core_axis_name="c", subcore_axis_name="s",
                               num_cores=NC, num_subcores=NS)
_CP = pltpu.CompilerParams(needs_layout_passes=False, use_tc_tiling_on_sc=False)


def _iota16():
    return lax.iota(jnp.int32, 16)


@functools.partial(
    pl.kernel, mesh=_MESH, compiler_params=_CP,
    out_type=jax.ShapeDtypeStruct((NP_ROWS, 128), jnp.float32),
    scratch_types=[
        pltpu.VMEM((NP_ROWS, 128), jnp.float32),
        pltpu.VMEM((CKD,), jnp.int32),
        pltpu.VMEM((CKD,), jnp.float32),
        pltpu.VMEM((NP_ROWS,), jnp.int32),
        pltpu.VMEM((25, 128), jnp.float32),
        pltpu.VMEM_SHARED((NP_ROWS, 128), jnp.float32),
    ],
)
def _deg_dinv_kernel(col_hbm, ew_hbm, dinv_hbm, hist, colb, ewb, ridx, dbuf,
                     deg_sp):
    c = lax.axis_index("c")
    s = lax.axis_index("s")
    iota = _iota16()

    def zr(g, _):
        fe = jnp.full((16,), g, jnp.int32)
        z = jnp.zeros((16,), jnp.float32)
        for h in range(8):
            plsc.store_scatter(hist, [fe, iota + 16 * h], z)
        return 0
    lax.fori_loop(0, NP_ROWS, zr, 0)

    def zi(g, _):
        v = iota + g * 16
        plsc.store_scatter(ridx, [v], v)
        return 0
    lax.fori_loop(0, NP_ROWS // 16, zi, 0)

    pltpu.sync_copy(hist.at[pl.ds(0, NP_ROWS // NS)],
                    deg_sp.at[pl.ds(s * (NP_ROWS // NS), NP_ROWS // NS)])
    plsc.subcore_barrier()

    def chunk(i, _):
        base = s * EPT + i * CKD
        pltpu.sync_copy(col_hbm.at[pl.ds(base, CKD)], colb)
        pltpu.sync_copy(ew_hbm.at[pl.ds(base, CKD)], ewb)

        def grp(g, _):
            cv = colb[pl.ds(g * 16, 16)]
            ev = ewb[pl.ds(g * 16, 16)]
            plsc.addupdate_scatter(hist, [cv >> 7, cv & 127], ev)
            return 0
        lax.fori_loop(0, CKD // 16, grp, 0)
        return 0
    lax.fori_loop(0, EPT // CKD, chunk, 0)

    pltpu.sync_copy(hist, deg_sp.at[ridx], add=True)
    plsc.subcore_barrier()

    r0 = c * (NP_ROWS // NC) + s * (NP_ROWS // NC // NS)
    pltpu.sync_copy(deg_sp.at[pl.ds(r0, 25)], dbuf)

    def nrow(g, _):
        fe = jnp.full((16,), g >> 3, jnp.int32)
        ln = ((g & 7) << 4) + iota
        x = plsc.load_gather(dbuf, [fe, ln]) + 1.0
        xi = plsc.bitcast(x, jnp.int32)
        y = plsc.bitcast(jnp.full((16,), 0x5F3759DF, jnp.int32) - (xi >> 1),
                        jnp.float32)
        hx = x * 0.5
        for _ in range(3):
            y = y * (1.5 - hx * y * y)
        plsc.store_scatter(dbuf, [fe, ln], y)
        return 0
    lax.fori_loop(0, 25 * 8, nrow, 0)
    pltpu.sync_copy(dbuf, dinv_hbm.at[pl.ds(r0, 25)])


@functools.partial(
    pl.kernel, mesh=_MESH, compiler_params=_CP,
    out_type=jax.ShapeDtypeStruct((N, H), jnp.float32),
    scratch_types=[
        pltpu.VMEM((2, CKA), jnp.int32),
        pltpu.VMEM((2, CKA), jnp.int32),
        pltpu.VMEM((2, CKA), jnp.float32),
        pltpu.VMEM((2, CKA, H), jnp.float32),
        pltpu.VMEM_SHARED((ACC_ROWS, H), jnp.float32),
        pltpu.SemaphoreType.DMA,
        pltpu.SemaphoreType.DMA,
        pltpu.SemaphoreType.DMA,
        pltpu.SemaphoreType.DMA,
    ],
)
def _agg_kernel(xs_hbm, row_hbm, col_hbm, ew_hbm, agg_hbm,
                rowb, lcb, nrmb, msg, acc_sp, isem0, isem1, gsem0, gsem1):
    c = lax.axis_index("c")
    s = lax.axis_index("s")
    iota = _iota16()
    lo = c * HALF
    isem = (isem0, isem1)
    gsem = (gsem0, gsem1)
    ebase = s * EPT
    nchunk = EPT // CKA

    def idx_copies(p, base):
        return (pltpu.make_async_copy(row_hbm.at[pl.ds(base, CKA)],
                                      rowb.at[p], isem[p]),
                pltpu.make_async_copy(col_hbm.at[pl.ds(base, CKA)],
                                      lcb.at[p], isem[p]),
                pltpu.make_async_copy(ew_hbm.at[pl.ds(base, CKA)],
                                      nrmb.at[p], isem[p]))

    def issue_idx(p, base):
        for d in idx_copies(p, base):
            d.start()

    def wait_idx(p, base):
        for d in idx_copies(p, base):
            d.wait()

    def issue_gather(p):
        return pltpu.async_copy(xs_hbm.at[rowb.at[p]], msg.at[p], gsem[p])

    def wait_gather(p):
        pltpu.make_async_copy(xs_hbm.at[rowb.at[p]], msg.at[p],
                              gsem[p]).wait()

    def lcg_loop(p):
        lp = lcb.at[p]

        def lcg(g, _):
            cv = lp[pl.ds(g * 16, 16)] - lo
            ok = (cv >= 0) & (cv < HALF)
            lp[pl.ds(g * 16, 16)] = jnp.where(ok, cv, HALF)
            return 0
        lax.fori_loop(0, CKA // 16, lcg, 0)

    def mul_loop(p):
        mp = msg.at[p]
        np_ = nrmb.at[p]

        def mul_grp(g, _):
            nvec = np_[pl.ds(g * 16, 16)]
            for j in range(16):
                e = g * 16 + j
                nv = lax.broadcast(nvec[j], (16,))
                mp[e, pl.ds(0, 16)] = mp[e, pl.ds(0, 16)] * nv
                mp[e, pl.ds(16, 16)] = mp[e, pl.ds(16, 16)] * nv
            return 0
        lax.fori_loop(0, CKA // 16, mul_grp, 0)

    def scatter(p):
        pltpu.sync_copy(msg.at[p], acc_sp.at[lcb.at[p]], add=True)

    def zr(g, _):
        fe = jnp.full((16,), g, jnp.int32)
        z = jnp.zeros((16,), jnp.float32)
        plsc.store_scatter(msg.at[0], [fe, iota], z)
        plsc.store_scatter(msg.at[0], [fe, iota + 16], z)
        return 0
    lax.fori_loop(0, CKA, zr, 0)
    zrows = ACC_ROWS // NS // 8
    for j in range(8):
        pltpu.sync_copy(
            msg.at[0].at[pl.ds(0, zrows)],
            acc_sp.at[pl.ds(s * (ACC_ROWS // NS) + j * zrows, zrows)])
    plsc.subcore_barrier()

    issue_idx(0, ebase)
    wait_idx(0, ebase)
    issue_gather(0)
    issue_idx(1, ebase + CKA)

    def body(ii, _):
        for p in range(2):
            i = 2 * ii + p
            base = ebase + i * CKA
            lcg_loop(p)
            wait_gather(p)
            wait_idx(1 - p, base + CKA)
            issue_gather(1 - p)
            mul_loop(p)
            scatter(p)
            issue_idx(p, base + 2 * CKA)
        return 0
    lax.fori_loop(0, (nchunk - 2) // 2, body, 0)

    lcg_loop(0)
    wait_gather(0)
    wait_idx(1, ebase + (nchunk - 1) * CKA)
    issue_gather(1)
    mul_loop(0)
    scatter(0)
    lcg_loop(1)
    wait_gather(1)
    mul_loop(1)
    scatter(1)
    plsc.subcore_barrier()

    rows_per_tile = HALF // NS
    sub = 125
    for j in range(rows_per_tile // sub):
        src = s * rows_per_tile + j * sub
        pltpu.sync_copy(acc_sp.at[pl.ds(src, sub)], msg.at[0].at[pl.ds(0, sub)])
        pltpu.sync_copy(msg.at[0].at[pl.ds(0, sub)],
                        agg_hbm.at[pl.ds(lo + src, sub)])


_RB = 2000


def _mm_in(x, W1, dinv_n):
    def body(x_ref, w_ref, d_ref, o_ref):
        o_ref[...] = d_ref[...] * jnp.dot(x_ref[...], w_ref[...],
                                          preferred_element_type=jnp.float32)
    return pl.pallas_call(
        body,
        grid=(N // _RB,),
        in_specs=[pl.BlockSpec((_RB, F_IN), lambda i: (i, 0)),
                  pl.BlockSpec((F_IN, H), lambda i: (0, 0)),
                  pl.BlockSpec((_RB, 1), lambda i: (i, 0))],
        out_specs=pl.BlockSpec((_RB, H), lambda i: (i, 0)),
        out_shape=jax.ShapeDtypeStruct((N, H), jnp.float32),
    )(x, W1, dinv_n)


def _mm_mid(agg, xs, dinv_n, b1, W2):
    def body(a_ref, x_ref, d_ref, b_ref, w_ref, o_ref):
        d = d_ref[...]
        h = jnp.maximum(d * (a_ref[...] + x_ref[...]) + b_ref[...], 0.0)
        o_ref[...] = d * jnp.dot(h, w_ref[...],
                                 preferred_element_type=jnp.float32)
    return pl.pallas_call(
        body,
        grid=(N // _RB,),
        in_specs=[pl.BlockSpec((_RB, H), lambda i: (i, 0)),
                  pl.BlockSpec((_RB, H), lambda i: (i, 0)),
                  pl.BlockSpec((_RB, 1), lambda i: (i, 0)),
                  pl.BlockSpec((1, H), lambda i: (0, 0)),
                  pl.BlockSpec((H, H), lambda i: (0, 0))],
        out_specs=pl.BlockSpec((_RB, H), lambda i: (i, 0)),
        out_shape=jax.ShapeDtypeStruct((N, H), jnp.float32),
    )(agg, xs, dinv_n, b1.reshape(1, H), W2)


def _mm_out(agg, xs, dinv_n, b2, Wfc, bfc):
    def body(a_ref, x_ref, d_ref, b_ref, w_ref, c_ref, o_ref):
        d = d_ref[...]
        h = jnp.maximum(d * (a_ref[...] + x_ref[...]) + b_ref[...], 0.0)
        o_ref[...] = jnp.dot(h, w_ref[...],
                             preferred_element_type=jnp.float32) + c_ref[...]
    return pl.pallas_call(
        body,
        grid=(N // _RB,),
        in_specs=[pl.BlockSpec((_RB, H), lambda i: (i, 0)),
                  pl.BlockSpec((_RB, H), lambda i: (i, 0)),
                  pl.BlockSpec((_RB, 1), lambda i: (i, 0)),
                  pl.BlockSpec((1, H), lambda i: (0, 0)),
                  pl.BlockSpec((H, 1), lambda i: (0, 0)),
                  pl.BlockSpec((1, 1), lambda i: (0, 0))],
        out_specs=pl.BlockSpec((_RB, 1), lambda i: (i, 0)),
        out_shape=jax.ShapeDtypeStruct((N, 1), jnp.float32),
    )(agg, xs, dinv_n, b2.reshape(1, H), Wfc, bfc.reshape(1, 1))


def kernel(x, c, ei, ew, W1, b1, W2, b2, Wfc, bfc):
    row = ei[0]
    col = ei[1]
    dinv = _deg_dinv_kernel(col, ew)
    dinv_n = dinv.reshape(-1)[:N].reshape(N, 1)

    xs1 = _mm_in(x, W1, dinv_n)
    agg1 = _agg_kernel(xs1, row, col, ew)
    xs2 = _mm_mid(agg1, xs1, dinv_n, b1, W2)
    agg2 = _agg_kernel(xs2, row, col, ew)
    return _mm_out(agg2, xs2, dinv_n, b2, Wfc, bfc)


# speedup vs baseline: 1.0071x; 1.0071x over previous
import functools
import jax
import jax.numpy as jnp
from jax import lax
from jax.experimental import pallas as pl
from jax.experimental.pallas import tpu as pltpu
from jax.experimental.pallas import tpu_sc as plsc

N = 100000
E = 1600000
F_IN = 128
H = 32

NC, NS, LANES = 2, 16, 16

NP_ROWS = 800
HALF = N // 2
ACC_ROWS = 50176
ZROWS = 784
EPT = E // NS
EPW = E // (NC * NS)
CK = 2000
CKA = 400
CKD = 2000

_MESH = plsc.VectorSubcoreMesh(core_axis_name="c", subcore_axis_name="s",
                               num_cores=NC, num_subcores=NS)
_CP = pltpu.CompilerParams(needs_layout_passes=False, use_tc_tiling_on_sc=False)


def _iota16():
    return lax.iota(jnp.int32, 16)


@functools.partial(
    pl.kernel, mesh=_MESH, compiler_params=_CP,
    out_type=jax.ShapeDtypeStruct((NP_ROWS, 128), jnp.float32),
    scratch_types=[
        pltpu.VMEM((NP_ROWS, 128), jnp.float32),
        pltpu.VMEM((CKD,), jnp.int32),
        pltpu.VMEM((CKD,), jnp.float32),
        pltpu.VMEM((NP_ROWS,), jnp.int32),
        pltpu.VMEM((25, 128), jnp.float32),
        pltpu.VMEM_SHARED((NP_ROWS, 128), jnp.float32),
    ],
)
def _deg_dinv_kernel(col_hbm, ew_hbm, dinv_hbm, hist, colb, ewb, ridx, dbuf,
                     deg_sp):
    c = lax.axis_index("c")
    s = lax.axis_index("s")
    iota = _iota16()

    def zr(g, _):
        fe = jnp.full((16,), g, jnp.int32)
        z = jnp.zeros((16,), jnp.float32)
        for h in range(8):
            plsc.store_scatter(hist, [fe, iota + 16 * h], z)
        return 0
    lax.fori_loop(0, NP_ROWS, zr, 0)

    def zi(g, _):
        v = iota + g * 16
        plsc.store_scatter(ridx, [v], v)
        return 0
    lax.fori_loop(0, NP_ROWS // 16, zi, 0)

    pltpu.sync_copy(hist.at[pl.ds(0, NP_ROWS // NS)],
                    deg_sp.at[pl.ds(s * (NP_ROWS // NS), NP_ROWS // NS)])
    plsc.subcore_barrier()

    def chunk(i, _):
        base = s * EPT + i * CKD
        pltpu.sync_copy(col_hbm.at[pl.ds(base, CKD)], colb)
        pltpu.sync_copy(ew_hbm.at[pl.ds(base, CKD)], ewb)

        def grp(g, _):
            cv = colb[pl.ds(g * 16, 16)]
            ev = ewb[pl.ds(g * 16, 16)]
            plsc.addupdate_scatter(hist, [cv >> 7, cv & 127], ev)
            return 0
        lax.fori_loop(0, CKD // 16, grp, 0)
        return 0
    lax.fori_loop(0, EPT // CKD, chunk, 0)

    pltpu.sync_copy(hist, deg_sp.at[ridx], add=True)
    plsc.subcore_barrier()

    r0 = c * (NP_ROWS // NC) + s * (NP_ROWS // NC // NS)
    pltpu.sync_copy(deg_sp.at[pl.ds(r0, 25)], dbuf)

    def nrow(g, _):
        fe = jnp.full((16,), g >> 3, jnp.int32)
        ln = ((g & 7) << 4) + iota
        x = plsc.load_gather(dbuf, [fe, ln]) + 1.0
        xi = plsc.bitcast(x, jnp.int32)
        y = plsc.bitcast(jnp.full((16,), 0x5F3759DF, jnp.int32) - (xi >> 1),
                        jnp.float32)
        hx = x * 0.5
        for _ in range(3):
            y = y * (1.5 - hx * y * y)
        plsc.store_scatter(dbuf, [fe, ln], y)
        return 0
    lax.fori_loop(0, 25 * 8, nrow, 0)
    pltpu.sync_copy(dbuf, dinv_hbm.at[pl.ds(r0, 25)])


@functools.partial(
    pl.kernel, mesh=_MESH, compiler_params=_CP,
    out_type=jax.ShapeDtypeStruct((N, H), jnp.float32),
    scratch_types=[
        pltpu.VMEM((2, CKA), jnp.int32),
        pltpu.VMEM((2, CKA), jnp.int32),
        pltpu.VMEM((2, CKA), jnp.float32),
        pltpu.VMEM((2, CKA, H), jnp.float32),
        pltpu.VMEM_SHARED((ACC_ROWS, H), jnp.float32),
        pltpu.SemaphoreType.DMA,
        pltpu.SemaphoreType.DMA,
        pltpu.SemaphoreType.DMA,
        pltpu.SemaphoreType.DMA,
    ],
)
def _agg_kernel(xs_hbm, row_hbm, col_hbm, ew_hbm, agg_hbm,
                rowb, lcb, nrmb, msg, acc_sp, isem0, isem1, gsem0, gsem1):
    c = lax.axis_index("c")
    s = lax.axis_index("s")
    iota = _iota16()
    lo = c * HALF
    isem = (isem0, isem1)
    gsem = (gsem0, gsem1)
    ebase = s * EPT
    nchunk = EPT // CKA

    def idx_copies(p, base):
        return (pltpu.make_async_copy(row_hbm.at[pl.ds(base, CKA)],
                                      rowb.at[p], isem[p]),
                pltpu.make_async_copy(col_hbm.at[pl.ds(base, CKA)],
                                      lcb.at[p], isem[p]),
                pltpu.make_async_copy(ew_hbm.at[pl.ds(base, CKA)],
                                      nrmb.at[p], isem[p]))

    def issue_idx(p, base):
        for d in idx_copies(p, base):
            d.start()

    def wait_idx(p, base):
        for d in idx_copies(p, base):
            d.wait()

    def issue_gather(p):
        return pltpu.async_copy(xs_hbm.at[rowb.at[p]], msg.at[p], gsem[p])

    def wait_gather(p):
        pltpu.make_async_copy(xs_hbm.at[rowb.at[p]], msg.at[p],
                              gsem[p]).wait()

    def lcg_loop(p):
        lp = lcb.at[p]

        def lcg(g, _):
            cv = lp[pl.ds(g * 16, 16)] - lo
            ok = (cv >= 0) & (cv < HALF)
            lp[pl.ds(g * 16, 16)] = jnp.where(ok, cv, HALF)
            return 0
        lax.fori_loop(0, CKA // 16, lcg, 0)

    def mul_loop(p):
        mp = msg.at[p]
        np_ = nrmb.at[p]

        def mul_grp(g, _):
            nvec = np_[pl.ds(g * 16, 16)]
            for j in range(16):
                fe = jnp.full((16,), g * 16 + j, jnp.int32)
                nv = lax.broadcast(nvec[j], (16,))
                a = plsc.load_gather(mp, [fe, iota])
                b = plsc.load_gather(mp, [fe, iota + 16])
                plsc.store_scatter(mp, [fe, iota], a * nv)
                plsc.store_scatter(mp, [fe, iota + 16], b * nv)
            return 0
        lax.fori_loop(0, CKA // 16, mul_grp, 0)

    def scatter(p):
        pltpu.sync_copy(msg.at[p], acc_sp.at[lcb.at[p]], add=True)

    def zr(g, _):
        fe = jnp.full((16,), g, jnp.int32)
        z = jnp.zeros((16,), jnp.float32)
        plsc.store_scatter(msg.at[0], [fe, iota], z)
        plsc.store_scatter(msg.at[0], [fe, iota + 16], z)
        return 0
    lax.fori_loop(0, CKA, zr, 0)
    zrows = ACC_ROWS // NS // 8
    for j in range(8):
        pltpu.sync_copy(
            msg.at[0].at[pl.ds(0, zrows)],
            acc_sp.at[pl.ds(s * (ACC_ROWS // NS) + j * zrows, zrows)])
    plsc.subcore_barrier()

    issue_idx(0, ebase)
    wait_idx(0, ebase)
    issue_gather(0)
    issue_idx(1, ebase + CKA)

    def body(ii, _):
        for p in range(2):
            i = 2 * ii + p
            base = ebase + i * CKA
            lcg_loop(p)
            wait_gather(p)
            wait_idx(1 - p, base + CKA)
            issue_gather(1 - p)
            mul_loop(p)
            scatter(p)
            issue_idx(p, base + 2 * CKA)
        return 0
    lax.fori_loop(0, (nchunk - 2) // 2, body, 0)

    lcg_loop(0)
    wait_gather(0)
    wait_idx(1, ebase + (nchunk - 1) * CKA)
    issue_gather(1)
    mul_loop(0)
    scatter(0)
    lcg_loop(1)
    wait_gather(1)
    mul_loop(1)
    scatter(1)
    plsc.subcore_barrier()

    rows_per_tile = HALF // NS
    sub = 125
    for j in range(rows_per_tile // sub):
        src = s * rows_per_tile + j * sub
        pltpu.sync_copy(acc_sp.at[pl.ds(src, sub)], msg.at[0].at[pl.ds(0, sub)])
        pltpu.sync_copy(msg.at[0].at[pl.ds(0, sub)],
                        agg_hbm.at[pl.ds(lo + src, sub)])


_RB = 2000


def _mm_in(x, W1, dinv_n):
    def body(x_ref, w_ref, d_ref, o_ref):
        o_ref[...] = d_ref[...] * jnp.dot(x_ref[...], w_ref[...],
                                          preferred_element_type=jnp.float32)
    return pl.pallas_call(
        body,
        grid=(N // _RB,),
        in_specs=[pl.BlockSpec((_RB, F_IN), lambda i: (i, 0)),
                  pl.BlockSpec((F_IN, H), lambda i: (0, 0)),
                  pl.BlockSpec((_RB, 1), lambda i: (i, 0))],
        out_specs=pl.BlockSpec((_RB, H), lambda i: (i, 0)),
        out_shape=jax.ShapeDtypeStruct((N, H), jnp.float32),
    )(x, W1, dinv_n)


def _mm_mid(agg, xs, dinv_n, b1, W2):
    def body(a_ref, x_ref, d_ref, b_ref, w_ref, o_ref):
        d = d_ref[...]
        h = jnp.maximum(d * (a_ref[...] + x_ref[...]) + b_ref[...], 0.0)
        o_ref[...] = d * jnp.dot(h, w_ref[...],
                                 preferred_element_type=jnp.float32)
    return pl.pallas_call(
        body,
        grid=(N // _RB,),
        in_specs=[pl.BlockSpec((_RB, H), lambda i: (i, 0)),
                  pl.BlockSpec((_RB, H), lambda i: (i, 0)),
                  pl.BlockSpec((_RB, 1), lambda i: (i, 0)),
                  pl.BlockSpec((1, H), lambda i: (0, 0)),
                  pl.BlockSpec((H, H), lambda i: (0, 0))],
        out_specs=pl.BlockSpec((_RB, H), lambda i: (i, 0)),
        out_shape=jax.ShapeDtypeStruct((N, H), jnp.float32),
    )(agg, xs, dinv_n, b1.reshape(1, H), W2)


def _mm_out(agg, xs, dinv_n, b2, Wfc, bfc):
    def body(a_ref, x_ref, d_ref, b_ref, w_ref, c_ref, o_ref):
        d = d_ref[...]
        h = jnp.maximum(d * (a_ref[...] + x_ref[...]) + b_ref[...], 0.0)
        o_ref[...] = jnp.dot(h, w_ref[...],
                             preferred_element_type=jnp.float32) + c_ref[...]
    return pl.pallas_call(
        body,
        grid=(N // _RB,),
        in_specs=[pl.BlockSpec((_RB, H), lambda i: (i, 0)),
                  pl.BlockSpec((_RB, H), lambda i: (i, 0)),
                  pl.BlockSpec((_RB, 1), lambda i: (i, 0)),
                  pl.BlockSpec((1, H), lambda i: (0, 0)),
                  pl.BlockSpec((H, 1), lambda i: (0, 0)),
                  pl.BlockSpec((1, 1), lambda i: (0, 0))],
        out_specs=pl.BlockSpec((_RB, 1), lambda i: (i, 0)),
        out_shape=jax.ShapeDtypeStruct((N, 1), jnp.float32),
    )(agg, xs, dinv_n, b2.reshape(1, H), Wfc, bfc.reshape(1, 1))


def kernel(x, c, ei, ew, W1, b1, W2, b2, Wfc, bfc):
    row = ei[0]
    col = ei[1]
    dinv = _deg_dinv_kernel(col, ew)
    dinv_n = dinv.reshape(-1)[:N].reshape(N, 1)

    xs1 = _mm_in(x, W1, dinv_n)
    agg1 = _agg_kernel(xs1, row, col, ew)
    xs2 = _mm_mid(agg1, xs1, dinv_n, b1, W2)
    agg2 = _agg_kernel(xs2, row, col, ew)
    return _mm_out(agg2, xs2, dinv_n, b2, Wfc, bfc)


# speedup vs baseline: 1.0351x; 1.0278x over previous
import functools
import jax
import jax.numpy as jnp
from jax import lax
from jax.experimental import pallas as pl
from jax.experimental.pallas import tpu as pltpu
from jax.experimental.pallas import tpu_sc as plsc

N = 100000
E = 1600000
F_IN = 128
H = 32

NC, NS, LANES = 2, 16, 16

NP_ROWS = 800
HALF = N // 2
ACC_ROWS = 50176
ZROWS = 784
EPT = E // NS
EPW = E // (NC * NS)
CK = 2000
CKA = 400
CKD = 2000

_MESH = plsc.VectorSubcoreMesh(core_axis_name="c", subcore_axis_name="s",
                               num_cores=NC, num_subcores=NS)
_CP = pltpu.CompilerParams(needs_layout_passes=False, use_tc_tiling_on_sc=False)


def _iota16():
    return lax.iota(jnp.int32, 16)


@functools.partial(
    pl.kernel, mesh=_MESH, compiler_params=_CP,
    out_type=jax.ShapeDtypeStruct((NP_ROWS, 128), jnp.float32),
    scratch_types=[
        pltpu.VMEM((NP_ROWS, 128), jnp.float32),
        pltpu.VMEM((CKD,), jnp.int32),
        pltpu.VMEM((CKD,), jnp.float32),
        pltpu.VMEM((NP_ROWS,), jnp.int32),
        pltpu.VMEM((25, 128), jnp.float32),
        pltpu.VMEM_SHARED((NP_ROWS, 128), jnp.float32),
    ],
)
def _deg_dinv_kernel(col_hbm, ew_hbm, dinv_hbm, hist, colb, ewb, ridx, dbuf,
                     deg_sp):
    c = lax.axis_index("c")
    s = lax.axis_index("s")
    iota = _iota16()

    def zr(g, _):
        fe = jnp.full((16,), g, jnp.int32)
        z = jnp.zeros((16,), jnp.float32)
        for h in range(8):
            plsc.store_scatter(hist, [fe, iota + 16 * h], z)
        return 0
    lax.fori_loop(0, NP_ROWS, zr, 0)

    def zi(g, _):
        v = iota + g * 16
        plsc.store_scatter(ridx, [v], v)
        return 0
    lax.fori_loop(0, NP_ROWS // 16, zi, 0)

    pltpu.sync_copy(hist.at[pl.ds(0, NP_ROWS // NS)],
                    deg_sp.at[pl.ds(s * (NP_ROWS // NS), NP_ROWS // NS)])
    plsc.subcore_barrier()

    def chunk(i, _):
        base = s * EPT + i * CKD
        pltpu.sync_copy(col_hbm.at[pl.ds(base, CKD)], colb)
        pltpu.sync_copy(ew_hbm.at[pl.ds(base, CKD)], ewb)

        def grp(g, _):
            cv = colb[pl.ds(g * 16, 16)]
            ev = ewb[pl.ds(g * 16, 16)]
            plsc.addupdate_scatter(hist, [cv >> 7, cv & 127], ev)
            return 0
        lax.fori_loop(0, CKD // 16, grp, 0)
        return 0
    lax.fori_loop(0, EPT // CKD, chunk, 0)

    pltpu.sync_copy(hist, deg_sp.at[ridx], add=True)
    plsc.subcore_barrier()

    r0 = c * (NP_ROWS // NC) + s * (NP_ROWS // NC // NS)
    pltpu.sync_copy(deg_sp.at[pl.ds(r0, 25)], dbuf)

    def nrow(g, _):
        fe = jnp.full((16,), g >> 3, jnp.int32)
        ln = ((g & 7) << 4) + iota
        x = plsc.load_gather(dbuf, [fe, ln]) + 1.0
        xi = plsc.bitcast(x, jnp.int32)
        y = plsc.bitcast(jnp.full((16,), 0x5F3759DF, jnp.int32) - (xi >> 1),
                        jnp.float32)
        hx = x * 0.5
        for _ in range(3):
            y = y * (1.5 - hx * y * y)
        plsc.store_scatter(dbuf, [fe, ln], y)
        return 0
    lax.fori_loop(0, 25 * 8, nrow, 0)
    pltpu.sync_copy(dbuf, dinv_hbm.at[pl.ds(r0, 25)])


@functools.partial(
    pl.kernel, mesh=_MESH, compiler_params=_CP,
    out_type=jax.ShapeDtypeStruct((N, H), jnp.float32),
    scratch_types=[
        pltpu.VMEM((2, CKA), jnp.int32),
        pltpu.VMEM((2, CKA), jnp.int32),
        pltpu.VMEM((2, CKA), jnp.int32),
        pltpu.VMEM((2, CKA), jnp.float32),
        pltpu.VMEM((2, CKA, H), jnp.float32),
        pltpu.VMEM_SHARED((ACC_ROWS, H), jnp.float32),
        pltpu.SemaphoreType.DMA,
        pltpu.SemaphoreType.DMA,
        pltpu.SemaphoreType.DMA,
        pltpu.SemaphoreType.DMA,
        pltpu.SemaphoreType.DMA,
        pltpu.SemaphoreType.DMA,
    ],
)
def _agg_kernel(xs_hbm, row_hbm, col_hbm, ew_hbm, agg_hbm,
                rowb, lcb, lcout, nrmb, msg, acc_sp,
                isem0, isem1, gsem0, gsem1, ssem0, ssem1):
    c = lax.axis_index("c")
    s = lax.axis_index("s")
    iota = _iota16()
    lo = c * HALF
    isem = (isem0, isem1)
    gsem = (gsem0, gsem1)
    ssem = (ssem0, ssem1)
    ebase = s * EPT
    nchunk = EPT // CKA

    def idx_copies(p, base):
        return (pltpu.make_async_copy(row_hbm.at[pl.ds(base, CKA)],
                                      rowb.at[p], isem[p]),
                pltpu.make_async_copy(col_hbm.at[pl.ds(base, CKA)],
                                      lcb.at[p], isem[p]),
                pltpu.make_async_copy(ew_hbm.at[pl.ds(base, CKA)],
                                      nrmb.at[p], isem[p]))

    def issue_idx(p, base):
        for d in idx_copies(p, base):
            d.start()

    def wait_idx(p, base):
        for d in idx_copies(p, base):
            d.wait()

    def issue_gather(p):
        pltpu.async_copy(xs_hbm.at[rowb.at[p]], msg.at[p], gsem[p])

    def wait_gather(p):
        pltpu.make_async_copy(xs_hbm.at[rowb.at[p]], msg.at[p],
                              gsem[p]).wait()

    def scatter_start(p):
        pltpu.async_copy(msg.at[p], acc_sp.at[lcout.at[p]], ssem[p],
                         add=True)

    def wait_scatter(p):
        pltpu.make_async_copy(msg.at[p], acc_sp.at[lcout.at[p]],
                              ssem[p]).wait()

    def lcg_loop(p):
        lp = lcb.at[p]
        op = lcout.at[p]

        def lcg(g, _):
            cv = lp[pl.ds(g * 16, 16)] - lo
            ok = (cv >= 0) & (cv < HALF)
            op[pl.ds(g * 16, 16)] = jnp.where(ok, cv, HALF)
            return 0
        lax.fori_loop(0, CKA // 16, lcg, 0)

    def mul_loop(p):
        mp = msg.at[p]
        np_ = nrmb.at[p]

        @functools.partial(plsc.parallel_loop, 0, CKA // 16, unroll=2)
        def mul_grp(g):
            nvec = np_[pl.ds(g * 16, 16)]
            for j in range(16):
                fe = jnp.full((16,), g * 16 + j, jnp.int32)
                nv = lax.broadcast(nvec[j], (16,))
                a = plsc.load_gather(mp, [fe, iota])
                b = plsc.load_gather(mp, [fe, iota + 16])
                plsc.store_scatter(mp, [fe, iota], a * nv)
                plsc.store_scatter(mp, [fe, iota + 16], b * nv)

    def zr(g, _):
        fe = jnp.full((16,), g, jnp.int32)
        z = jnp.zeros((16,), jnp.float32)
        plsc.store_scatter(msg.at[0], [fe, iota], z)
        plsc.store_scatter(msg.at[0], [fe, iota + 16], z)
        return 0
    lax.fori_loop(0, CKA, zr, 0)
    zrows = ACC_ROWS // NS // 8
    for j in range(8):
        pltpu.sync_copy(
            msg.at[0].at[pl.ds(0, zrows)],
            acc_sp.at[pl.ds(s * (ACC_ROWS // NS) + j * zrows, zrows)])
    plsc.subcore_barrier()

    def chunk_step(base, q, first=False, issue_g=True, issue_i=True):
        lcg_loop(q)
        wait_gather(q)
        if not first:
            wait_scatter(1 - q)
        if issue_g:
            wait_idx(1 - q, base + CKA)
            issue_gather(1 - q)
        mul_loop(q)
        scatter_start(q)
        if issue_i:
            issue_idx(q, base + 2 * CKA)

    issue_idx(0, ebase)
    wait_idx(0, ebase)
    issue_gather(0)
    issue_idx(1, ebase + CKA)
    chunk_step(ebase, 0, first=True)

    def body(ii, _):
        b1 = ebase + (2 * ii + 1) * CKA
        chunk_step(b1, 1)
        chunk_step(b1 + CKA, 0)
        return 0
    lax.fori_loop(0, (nchunk - 4) // 2, body, 0)

    chunk_step(ebase + (nchunk - 3) * CKA, 1)
    chunk_step(ebase + (nchunk - 2) * CKA, 0, issue_i=False)
    chunk_step(ebase + (nchunk - 1) * CKA, 1,
               issue_g=False, issue_i=False)
    wait_scatter(1)
    plsc.subcore_barrier()

    rows_per_tile = HALF // NS
    sub = 125
    for j in range(rows_per_tile // sub):
        src = s * rows_per_tile + j * sub
        pltpu.sync_copy(acc_sp.at[pl.ds(src, sub)], msg.at[0].at[pl.ds(0, sub)])
        pltpu.sync_copy(msg.at[0].at[pl.ds(0, sub)],
                        agg_hbm.at[pl.ds(lo + src, sub)])


_RB = 2000


def _mm_in(x, W1, dinv_n):
    def body(x_ref, w_ref, d_ref, o_ref):
        o_ref[...] = d_ref[...] * jnp.dot(x_ref[...], w_ref[...],
                                          preferred_element_type=jnp.float32)
    return pl.pallas_call(
        body,
        grid=(N // _RB,),
        in_specs=[pl.BlockSpec((_RB, F_IN), lambda i: (i, 0)),
                  pl.BlockSpec((F_IN, H), lambda i: (0, 0)),
                  pl.BlockSpec((_RB, 1), lambda i: (i, 0))],
        out_specs=pl.BlockSpec((_RB, H), lambda i: (i, 0)),
        out_shape=jax.ShapeDtypeStruct((N, H), jnp.float32),
    )(x, W1, dinv_n)


def _mm_mid(agg, xs, dinv_n, b1, W2):
    def body(a_ref, x_ref, d_ref, b_ref, w_ref, o_ref):
        d = d_ref[...]
        h = jnp.maximum(d * (a_ref[...] + x_ref[...]) + b_ref[...], 0.0)
        o_ref[...] = d * jnp.dot(h, w_ref[...],
                                 preferred_element_type=jnp.float32)
    return pl.pallas_call(
        body,
        grid=(N // _RB,),
        in_specs=[pl.BlockSpec((_RB, H), lambda i: (i, 0)),
                  pl.BlockSpec((_RB, H), lambda i: (i, 0)),
                  pl.BlockSpec((_RB, 1), lambda i: (i, 0)),
                  pl.BlockSpec((1, H), lambda i: (0, 0)),
                  pl.BlockSpec((H, H), lambda i: (0, 0))],
        out_specs=pl.BlockSpec((_RB, H), lambda i: (i, 0)),
        out_shape=jax.ShapeDtypeStruct((N, H), jnp.float32),
    )(agg, xs, dinv_n, b1.reshape(1, H), W2)


def _mm_out(agg, xs, dinv_n, b2, Wfc, bfc):
    def body(a_ref, x_ref, d_ref, b_ref, w_ref, c_ref, o_ref):
        d = d_ref[...]
        h = jnp.maximum(d * (a_ref[...] + x_ref[...]) + b_ref[...], 0.0)
        o_ref[...] = jnp.dot(h, w_ref[...],
                             preferred_element_type=jnp.float32) + c_ref[...]
    return pl.pallas_call(
        body,
        grid=(N // _RB,),
        in_specs=[pl.BlockSpec((_RB, H), lambda i: (i, 0)),
                  pl.BlockSpec((_RB, H), lambda i: (i, 0)),
                  pl.BlockSpec((_RB, 1), lambda i: (i, 0)),
                  pl.BlockSpec((1, H), lambda i: (0, 0)),
                  pl.BlockSpec((H, 1), lambda i: (0, 0)),
                  pl.BlockSpec((1, 1), lambda i: (0, 0))],
        out_specs=pl.BlockSpec((_RB, 1), lambda i: (i, 0)),
        out_shape=jax.ShapeDtypeStruct((N, 1), jnp.float32),
    )(agg, xs, dinv_n, b2.reshape(1, H), Wfc, bfc.reshape(1, 1))


def kernel(x, c, ei, ew, W1, b1, W2, b2, Wfc, bfc):
    row = ei[0]
    col = ei[1]
    dinv = _deg_dinv_kernel(col, ew)
    dinv_n = dinv.reshape(-1)[:N].reshape(N, 1)

    xs1 = _mm_in(x, W1, dinv_n)
    agg1 = _agg_kernel(xs1, row, col, ew)
    xs2 = _mm_mid(agg1, xs1, dinv_n, b1, W2)
    agg2 = _agg_kernel(xs2, row, col, ew)
    return _mm_out(agg2, xs2, dinv_n, b2, Wfc, bfc)
